# baseline (speedup 1.0000x reference)
import functools

import jax
import jax.numpy as jnp
from jax import lax
from jax.experimental import pallas as pl
from jax.experimental.pallas import tpu as pltpu
from jax.experimental.pallas import tpu_sc as plsc

_MAX_ROWS = 8192
_EMB = 2048
_OUT_ROWS = 4096

_NC, _NS = 2, 16
_NW = _NC * _NS
_RPW = _OUT_ROWS // _NW
_CHUNK = 32
_NCHUNK = _RPW // _CHUNK

_mesh = plsc.VectorSubcoreMesh(
    core_axis_name="c", subcore_axis_name="s",
    num_cores=_NC, num_subcores=_NS,
)


@functools.partial(
    pl.kernel,
    mesh=_mesh,
    out_type=jax.ShapeDtypeStruct((_OUT_ROWS, _EMB), jnp.float32),
    scratch_types=[
        pltpu.VMEM((_CHUNK,), jnp.int32),
        pltpu.VMEM((_CHUNK, _EMB), jnp.float32),
        pltpu.SemaphoreType.DMA,
    ],
)
def _gather_rows(table_hbm, idx_hbm, out_hbm, idx_v, rows_v, sem):
    wid = lax.axis_index("s") * _NC + lax.axis_index("c")
    base = wid * _RPW
    for j in range(_NCHUNK):
        off = base + j * _CHUNK
        pltpu.sync_copy(idx_hbm.at[pl.ds(off, _CHUNK)], idx_v)
        pltpu.async_copy(table_hbm.at[idx_v], rows_v, sem).wait()
        pltpu.sync_copy(rows_v, out_hbm.at[pl.ds(off, _CHUNK)])


def kernel(seq_len, start_pos, pos_embeddings):
    start_row = (jnp.asarray(start_pos, jnp.int32)
                 + jnp.asarray(seq_len, jnp.int32) - _OUT_ROWS)
    start_row = jnp.clip(start_row, 0, _MAX_ROWS - _OUT_ROWS)
    row_idx = start_row + lax.iota(jnp.int32, _OUT_ROWS)
    return _gather_rows(pos_embeddings, row_idx)
